# Initial kernel scaffold; baseline (speedup 1.0000x reference)
import jax
import jax.numpy as jnp
from jax.experimental import pallas as pl


def kernel(input, target):
    raise NotImplementedError("write your pallas kernel here")


# speedup vs baseline: 1.3855x; 1.3855x over previous
import functools

import jax
import jax.numpy as jnp
from jax import lax
from jax.experimental import pallas as pl
from jax.experimental.pallas import tpu as pltpu
from jax.experimental.pallas import tpu_sc as plsc

_N = 16384
_C = 340
_NC = 2
_NS = 16
_L = 16
_NW = _NC * _NS
_RPW = _N // _NW
_CHUNK = 128
_NCHUNK = _RPW // _CHUNK
_CW = _C * _CHUNK
_GPC = _CHUNK // _L


def _tree_sum(vs):
    vs = list(vs)
    while len(vs) > 1:
        nxt = [vs[i] + vs[i + 1] for i in range(0, len(vs) - 1, 2)]
        if len(vs) % 2:
            nxt.append(vs[-1])
        vs = nxt
    return vs[0]


def _sc_body(x_hbm, t_hbm, out_hbm, xb, tv, idxv, xtb, rsb, ov, sem):
    wid = lax.axis_index("s") * _NC + lax.axis_index("c")
    lane = lax.broadcasted_iota(jnp.int32, (_L,), 0)
    tail_mask = lane >= 12

    pltpu.sync_copy(t_hbm.at[pl.ds(wid * _RPW, _RPW)], tv)

    def idx_body(gg, carry):
        tvec = tv[pl.ds(gg * _L, _L)]
        idxv[pl.ds(gg * _L, _L)] = (wid * _RPW + gg * _L + lane) * _C + tvec
        return carry

    lax.fori_loop(0, _RPW // _L, idx_body, 0, unroll=False)
    gather = pltpu.async_copy(x_hbm.at[idxv], xtb, sem)

    def chunk_body(c, carry):
        base = wid * (_RPW * _C) + c * _CW
        pltpu.sync_copy(x_hbm.at[pl.ds(base, _CW)], xb)

        def group_body(g, carry2):
            parts = []
            for rr in range(_L):
                rb = (g * _L + rr) * _C
                es = [jnp.exp(xb[pl.ds(rb + 16 * k, 16)]) for k in range(21)]
                vt = xb[pl.ds(rb + 324, 16)]
                es.append(jnp.where(tail_mask, jnp.exp(vt), 0.0))
                s = jnp.sum(_tree_sum(es))
                parts.append(
                    jnp.where(lane == rr, jnp.broadcast_to(s, (_L,)), 0.0)
                )
            rsb[pl.ds((c * _GPC + g) * _L, _L)] = _tree_sum(parts)
            return carry2

        return lax.fori_loop(0, _GPC, group_body, carry, unroll=False)

    lax.fori_loop(0, _NCHUNK, chunk_body, 0, unroll=False)

    gather.wait()
    total = jnp.zeros((_L,), jnp.float32)
    for gg in range(_RPW // _L):
        xtv = xtb[pl.ds(gg * _L, _L)]
        rs = rsb[pl.ds(gg * _L, _L)]
        total = total + jnp.exp(-xtv) * (rs - jnp.exp(xtv))
    ov[...] = total
    pltpu.sync_copy(ov, out_hbm.at[wid])


@jax.jit
def kernel(input, target):
    x_flat = input.reshape(-1)
    t32 = target.astype(jnp.int32)
    run = functools.partial(
        pl.kernel,
        mesh=plsc.VectorSubcoreMesh(core_axis_name="c", subcore_axis_name="s"),
        compiler_params=pltpu.CompilerParams(needs_layout_passes=False),
        out_type=jax.ShapeDtypeStruct((_NW, _L), jnp.float32),
        scratch_types=[
            pltpu.VMEM((_CW,), jnp.float32),
            pltpu.VMEM((_RPW,), jnp.int32),
            pltpu.VMEM((_RPW,), jnp.int32),
            pltpu.VMEM((_RPW,), jnp.float32),
            pltpu.VMEM((_RPW,), jnp.float32),
            pltpu.VMEM((_L,), jnp.float32),
            pltpu.SemaphoreType.DMA,
        ],
    )(_sc_body)
    parts = run(x_flat, t32)
    return jnp.log(1.0 + jnp.sum(parts))


# speedup vs baseline: 2.1887x; 1.5797x over previous
import functools

import jax
import jax.numpy as jnp
from jax import lax
from jax.experimental import pallas as pl
from jax.experimental.pallas import tpu as pltpu
from jax.experimental.pallas import tpu_sc as plsc

_N = 16384
_C = 340
_NC = 2
_NS = 16
_L = 16
_NW = _NC * _NS
_RPW = _N // _NW
_SR = 8
_NFS = (_C // _SR) // 2 * 2
_REM = _C - _NFS * _SR
_NG = _RPW // _L


def _tree_sum(vs):
    vs = list(vs)
    while len(vs) > 1:
        nxt = [vs[i] + vs[i + 1] for i in range(0, len(vs) - 1, 2)]
        if len(vs) % 2:
            nxt.append(vs[-1])
        vs = nxt
    return vs[0]


def _sc_body(x_hbm, t_hbm, out_hbm, xb0, xb1, xbr, tvb, rsb, xeb, ov,
             sem0, sem1):
    wid = lax.axis_index("s") * _NC + lax.axis_index("c")
    base = wid * _RPW

    pltpu.sync_copy(t_hbm.at[pl.ds(base, _RPW)], tvb)

    zero = jnp.zeros((_L,), jnp.float32)

    def zero_body(g, c):
        rsb[pl.ds(g * _L, _L)] = zero
        xeb[pl.ds(g * _L, _L)] = zero
        return c

    lax.fori_loop(0, _NG, zero_body, 0, unroll=False)

    def stripe_src(s):
        return x_hbm.at[pl.ds(s * _SR, _SR), pl.ds(base, _RPW)]

    def compute(buf, s, nrows):
        def gbody(g, c):
            col = g * _L
            tl = tvb[pl.ds(col, _L)] - s * _SR
            evs = []
            xts = []
            for jr in range(nrows):
                ev = jnp.exp(buf[jr, pl.ds(col, _L)])
                evs.append(ev)
                xts.append(jnp.where(tl == jr, ev, 0.0))
            rsb[pl.ds(col, _L)] += _tree_sum(evs)
            xeb[pl.ds(col, _L)] += _tree_sum(xts)
            return c

        lax.fori_loop(0, _NG, gbody, 0, unroll=False)

    pltpu.async_copy(stripe_src(0), xb0, sem0)

    def pair_body(p, c):
        s0 = p * 2
        s1 = s0 + 1
        pltpu.async_copy(stripe_src(s1), xb1, sem1)
        pltpu.make_async_copy(stripe_src(s0), xb0, sem0).wait()
        compute(xb0, s0, _SR)

        @pl.when(p < _NFS // 2 - 1)
        def _():
            pltpu.async_copy(stripe_src(s0 + 2), xb0, sem0)

        pltpu.make_async_copy(stripe_src(s1), xb1, sem1).wait()
        compute(xb1, s1, _SR)
        return c

    lax.fori_loop(0, _NFS // 2, pair_body, 0, unroll=False)

    pltpu.sync_copy(
        x_hbm.at[pl.ds(_NFS * _SR, _REM), pl.ds(base, _RPW)], xbr
    )
    compute(xbr, _NFS, _REM)

    total = jnp.zeros((_L,), jnp.float32)
    for g in range(_NG):
        rs = rsb[pl.ds(g * _L, _L)]
        xe = xeb[pl.ds(g * _L, _L)]
        total = total + (rs - xe) / xe
    ov[...] = total
    pltpu.sync_copy(ov, out_hbm.at[wid])


@jax.jit
def kernel(input, target):
    xt = input.T
    t32 = target.astype(jnp.int32)
    run = functools.partial(
        pl.kernel,
        mesh=plsc.VectorSubcoreMesh(core_axis_name="c", subcore_axis_name="s"),
        compiler_params=pltpu.CompilerParams(needs_layout_passes=False),
        out_type=jax.ShapeDtypeStruct((_NW, _L), jnp.float32),
        scratch_types=[
            pltpu.VMEM((_SR, _RPW), jnp.float32),
            pltpu.VMEM((_SR, _RPW), jnp.float32),
            pltpu.VMEM((_REM, _RPW), jnp.float32),
            pltpu.VMEM((_RPW,), jnp.int32),
            pltpu.VMEM((_RPW,), jnp.float32),
            pltpu.VMEM((_RPW,), jnp.float32),
            pltpu.VMEM((_L,), jnp.float32),
            pltpu.SemaphoreType.DMA,
            pltpu.SemaphoreType.DMA,
        ],
    )(_sc_body)
    parts = run(xt, t32)
    return jnp.log(1.0 + jnp.sum(parts))


# speedup vs baseline: 3.0594x; 1.3978x over previous
import functools

import jax
import jax.numpy as jnp
from jax import lax
from jax.experimental import pallas as pl
from jax.experimental.pallas import tpu as pltpu
from jax.experimental.pallas import tpu_sc as plsc

_N = 16384
_C = 340
_NC = 2
_NS = 16
_L = 16
_NW = _NC * _NS
_RPW = _N // _NW
_CK = 48
_NCK = _C // _CK
_REM = _C - _NCK * _CK
_NG = _RPW // _L


def _tree_sum(vs):
    vs = list(vs)
    while len(vs) > 1:
        nxt = [vs[i] + vs[i + 1] for i in range(0, len(vs) - 1, 2)]
        if len(vs) % 2:
            nxt.append(vs[-1])
        vs = nxt
    return vs[0]


def _sc_body(x_hbm, t_hbm, out_hbm, xb0, xb1, xbr, tvb, rsb, xeb, ov,
             sem0, sem1):
    wid = lax.axis_index("s") * _NC + lax.axis_index("c")
    base = wid * _RPW
    lane = lax.broadcasted_iota(jnp.int32, (_L,), 0)

    pltpu.sync_copy(t_hbm.at[pl.ds(base, _RPW)], tvb)

    zero = jnp.zeros((_L,), jnp.float32)

    def zero_body(g, c):
        rsb[pl.ds(g * _L, _L)] = zero
        xeb[pl.ds(g * _L, _L)] = zero
        return c

    lax.fori_loop(0, _NG, zero_body, 0, unroll=False)

    def chunk_src(s, nrows=_CK):
        return x_hbm.at[pl.ds(s * _CK, nrows), pl.ds(base, _RPW)]

    def compute(buf, s, nrows):
        def gbody(g, c):
            col = g * _L
            tl = tvb[pl.ds(col, _L)] - s * _CK
            evs = []
            for jr in range(nrows):
                evs.append(jnp.exp(buf[jr, pl.ds(col, _L)]))
            rsb[pl.ds(col, _L)] += _tree_sum(evs)
            m = (tl >= 0) & (tl < nrows)
            tr = jnp.clip(tl, 0, nrows - 1)
            xg = plsc.load_gather(buf, [tr, col + lane])
            xeb[pl.ds(col, _L)] += jnp.where(m, jnp.exp(xg), 0.0)
            return c

        lax.fori_loop(0, _NG, gbody, 0, unroll=False)

    pltpu.async_copy(chunk_src(0), xb0, sem0)

    def pair_body(p, c):
        s0 = p * 2
        s1 = s0 + 1
        pltpu.async_copy(chunk_src(s1), xb1, sem1)
        pltpu.make_async_copy(chunk_src(s0), xb0, sem0).wait()
        compute(xb0, s0, _CK)
        pltpu.async_copy(chunk_src(s0 + 2), xb0, sem0)
        pltpu.make_async_copy(chunk_src(s1), xb1, sem1).wait()
        compute(xb1, s1, _CK)
        return c

    lax.fori_loop(0, (_NCK - 1) // 2, pair_body, 0, unroll=False)

    pltpu.make_async_copy(chunk_src(_NCK - 1), xb0, sem0).wait()
    compute(xb0, _NCK - 1, _CK)

    pltpu.sync_copy(chunk_src(_NCK, _REM), xbr)
    compute(xbr, _NCK, _REM)

    total = jnp.zeros((_L,), jnp.float32)
    for g in range(_NG):
        rs = rsb[pl.ds(g * _L, _L)]
        xe = xeb[pl.ds(g * _L, _L)]
        total = total + (rs - xe) / xe
    ov[...] = total
    pltpu.sync_copy(ov, out_hbm.at[wid])


@jax.jit
def kernel(input, target):
    xt = input.T
    t32 = target.astype(jnp.int32)
    run = functools.partial(
        pl.kernel,
        mesh=plsc.VectorSubcoreMesh(core_axis_name="c", subcore_axis_name="s"),
        compiler_params=pltpu.CompilerParams(needs_layout_passes=False),
        out_type=jax.ShapeDtypeStruct((_NW, _L), jnp.float32),
        scratch_types=[
            pltpu.VMEM((_CK, _RPW), jnp.float32),
            pltpu.VMEM((_CK, _RPW), jnp.float32),
            pltpu.VMEM((_REM, _RPW), jnp.float32),
            pltpu.VMEM((_RPW,), jnp.int32),
            pltpu.VMEM((_RPW,), jnp.float32),
            pltpu.VMEM((_RPW,), jnp.float32),
            pltpu.VMEM((_L,), jnp.float32),
            pltpu.SemaphoreType.DMA,
            pltpu.SemaphoreType.DMA,
        ],
    )(_sc_body)
    parts = run(xt, t32)
    return jnp.log(1.0 + jnp.sum(parts))
